# baseline (speedup 1.0000x reference)
import functools

import jax
import jax.numpy as jnp
from jax import lax
from jax.experimental import pallas as pl
from jax.experimental.pallas import tpu as pltpu
from jax.experimental.pallas import tpu_sc as plsc

NC, NS, L = 2, 16, 16
NW = NC * NS
D = 128
CHUNK = 128
N_PAD = 10240
BM = 512


def _make_deg_kernel(e_pad):
    per_w = e_pad // NW
    nch = per_w // CHUNK
    mesh = plsc.VectorSubcoreMesh(core_axis_name="c", subcore_axis_name="s")

    @functools.partial(
        pl.kernel,
        mesh=mesh,
        out_type=jax.ShapeDtypeStruct((NW, N_PAD), jnp.float32),
        compiler_params=pltpu.CompilerParams(needs_layout_passes=False),
        scratch_types=[
            pltpu.VMEM((nch, CHUNK), jnp.int32),
            pltpu.VMEM((N_PAD,), jnp.float32),
        ],
    )
    def deg_kernel(dst_hbm, out_hbm, idx_v, hist_v):
        c = lax.axis_index("c")
        s = lax.axis_index("s")
        wid = c * NS + s

        zeros16 = jnp.zeros((L,), jnp.float32)
        ones16 = jnp.ones((L,), jnp.float32)

        pltpu.sync_copy(dst_hbm.at[wid], idx_v)

        def zero_body(i, _):
            hist_v[pl.ds(pl.multiple_of(i * L, L), L)] = zeros16
            return 0

        lax.fori_loop(0, N_PAD // L, zero_body, 0)

        def chunk_body(j, _):
            for k in range(CHUNK // L):
                didx = idx_v[j, pl.ds(k * L, L)]
                plsc.addupdate_scatter(hist_v, [didx], ones16)
            return 0

        lax.fori_loop(0, nch, chunk_body, 0)
        pltpu.sync_copy(hist_v, out_hbm.at[wid])

    return deg_kernel


def _make_edge_kernel(e_pad):
    per_w = e_pad // NW
    nch = per_w // CHUNK
    ib = 16
    assert nch % ib == 0
    nblk = nch // ib
    zr = 16
    assert (N_PAD // NS) % zr == 0
    mesh = plsc.VectorSubcoreMesh(core_axis_name="c", subcore_axis_name="s")

    @functools.partial(
        pl.kernel,
        mesh=mesh,
        out_type=jax.ShapeDtypeStruct((NC, N_PAD, D), jnp.float32),
        compiler_params=pltpu.CompilerParams(needs_layout_passes=False),
        scratch_types=[
            pltpu.VMEM((2, ib, CHUNK), jnp.int32),
            pltpu.VMEM((2, ib, CHUNK), jnp.int32),
            pltpu.VMEM((4, CHUNK // 2, D), jnp.float32),
            pltpu.VMEM((zr, D), jnp.float32),
            pltpu.VMEM_SHARED((N_PAD, D), jnp.float32),
            pltpu.SemaphoreType.DMA,
            pltpu.SemaphoreType.DMA,
            pltpu.SemaphoreType.DMA,
            pltpu.SemaphoreType.DMA,
            pltpu.SemaphoreType.DMA,
        ],
    )
    def edge_kernel(src_hbm, dst_hbm, g_hbm, out_hbm,
                    sidx, didx, rows, zrow, acc,
                    sem0, sem1, sem2, sem3, isem):
        sems = (sem0, sem1, sem2, sem3)
        c = lax.axis_index("c")
        s = lax.axis_index("s")
        wid = c * NS + s

        pltpu.sync_copy(src_hbm.at[wid, 0], sidx.at[0])
        pltpu.sync_copy(dst_hbm.at[wid, 0], didx.at[0])

        zeros16 = jnp.zeros((L,), jnp.float32)

        def zrow_body(r, _):
            for k in range(D // L):
                zrow[r, pl.ds(k * L, L)] = zeros16
            return 0

        lax.fori_loop(0, zr, zrow_body, 0)

        rows_per_tile = N_PAD // NS

        def zcopy_body(i, _):
            off = pl.multiple_of(s * rows_per_tile + i * zr, zr)
            pltpu.sync_copy(zrow, acc.at[pl.ds(off, zr)])
            return 0

        lax.fori_loop(0, rows_per_tile // zr, zcopy_body, 0)
        plsc.subcore_barrier()

        def blk_body(n, _):
            bb = lax.rem(n, 2)

            @pl.when(n + 1 < nblk)
            def _():
                pltpu.async_copy(src_hbm.at[wid, n + 1], sidx.at[1 - bb],
                                 isem)
                pltpu.async_copy(dst_hbm.at[wid, n + 1], didx.at[1 - bb],
                                 isem)

            H = CHUNK // 2
            nsub = 2 * ib

            def gidx(bb2, j4, b):
                cj = j4 * 2 + b // 2
                off = (b % 2) * H
                return (bb2, cj, pl.ds(off, H))

            for p in range(3):
                pltpu.async_copy(g_hbm.at[sidx.at[gidx(bb, 0, p)]],
                                 rows.at[p], sems[p])

            def quad_body(j4, _):
                for b in range(4):
                    j = j4 * 4 + b
                    nb = (b + 3) % 4

                    pltpu.make_async_copy(
                        g_hbm.at[sidx.at[gidx(bb, j4, b)]],
                        rows.at[b], sems[b]).wait()

                    @pl.when(j + 3 < nsub)
                    def _():
                        pltpu.async_copy(
                            g_hbm.at[sidx.at[gidx(bb, j4, b + 3)]],
                            rows.at[nb], sems[nb])

                    pltpu.sync_copy(rows.at[b],
                                    acc.at[didx.at[gidx(bb, j4, b)]],
                                    add=True)
                return 0

            lax.fori_loop(0, nsub // 4, quad_body, 0)

            @pl.when(n + 1 < nblk)
            def _():
                pltpu.make_async_copy(src_hbm.at[wid, n + 1],
                                      sidx.at[1 - bb], isem).wait()
                pltpu.make_async_copy(dst_hbm.at[wid, n + 1],
                                      didx.at[1 - bb], isem).wait()
            return 0

        lax.fori_loop(0, nblk, blk_body, 0)
        plsc.subcore_barrier()

        out_off = pl.multiple_of(s * rows_per_tile, rows_per_tile)
        pltpu.sync_copy(acc.at[pl.ds(out_off, rows_per_tile)],
                        out_hbm.at[c, pl.ds(out_off, rows_per_tile)])

    return edge_kernel


def _mm_scale_body(deg_ref, x_ref, w_ref, g_ref, dinv_ref):
    deg = jnp.sum(deg_ref[...], axis=0) + 1.0
    dinv = lax.rsqrt(deg)
    h = jnp.dot(x_ref[...], w_ref[...], preferred_element_type=jnp.float32)
    g_ref[...] = h * dinv[:, None]
    dinv_ref[...] = dinv


def _mm_scale(deg_parts, x, w):
    grid = (N_PAD // BM,)
    return pl.pallas_call(
        _mm_scale_body,
        grid=grid,
        in_specs=[
            pl.BlockSpec((NW, BM), lambda i: (0, i)),
            pl.BlockSpec((BM, D), lambda i: (i, 0)),
            pl.BlockSpec((D, D), lambda i: (0, 0)),
        ],
        out_specs=[
            pl.BlockSpec((BM, D), lambda i: (i, 0)),
            pl.BlockSpec((BM,), lambda i: (i,)),
        ],
        out_shape=[
            jax.ShapeDtypeStruct((N_PAD, D), jnp.float32),
            jax.ShapeDtypeStruct((N_PAD,), jnp.float32),
        ],
    )(deg_parts, x, w)


def _mid_body(n, a0_ref, a1_ref, g_ref, dinv_ref, b_ref, w_ref, g2_ref):
    dinv = dinv_ref[...]
    acc = a0_ref[...] + a1_ref[...] + g_ref[...]
    h = jnp.maximum(acc * dinv[:, None] + b_ref[...], 0.0)
    g2 = jnp.dot(h, w_ref[...],
                 preferred_element_type=jnp.float32) * dinv[:, None]
    row = (pl.program_id(0) * BM
           + lax.broadcasted_iota(jnp.int32, (BM, D), 0))
    g2_ref[...] = jnp.where(row < n, g2, 0.0)


def _mid(a0, a1, g, dinv, b, w, n):
    grid = (N_PAD // BM,)
    return pl.pallas_call(
        functools.partial(_mid_body, n),
        grid=grid,
        in_specs=[
            pl.BlockSpec((BM, D), lambda i: (i, 0)),
            pl.BlockSpec((BM, D), lambda i: (i, 0)),
            pl.BlockSpec((BM, D), lambda i: (i, 0)),
            pl.BlockSpec((BM,), lambda i: (i,)),
            pl.BlockSpec((1, D), lambda i: (0, 0)),
            pl.BlockSpec((D, D), lambda i: (0, 0)),
        ],
        out_specs=pl.BlockSpec((BM, D), lambda i: (i, 0)),
        out_shape=jax.ShapeDtypeStruct((N_PAD, D), jnp.float32),
    )(a0, a1, g, dinv, b, w)


def _final_body(a0_ref, a1_ref, g_ref, dinv_ref, b_ref, out_ref):
    dinv = dinv_ref[...]
    acc = a0_ref[...] + a1_ref[...] + g_ref[...]
    out_ref[...] = acc * dinv[:, None] + b_ref[...]


def _final(a0, a1, g, dinv, b):
    grid = (N_PAD // BM,)
    return pl.pallas_call(
        _final_body,
        grid=grid,
        in_specs=[
            pl.BlockSpec((BM, D), lambda i: (i, 0)),
            pl.BlockSpec((BM, D), lambda i: (i, 0)),
            pl.BlockSpec((BM, D), lambda i: (i, 0)),
            pl.BlockSpec((BM,), lambda i: (i,)),
            pl.BlockSpec((1, D), lambda i: (0, 0)),
        ],
        out_specs=pl.BlockSpec((BM, D), lambda i: (i, 0)),
        out_shape=jax.ShapeDtypeStruct((N_PAD, D), jnp.float32),
    )(a0, a1, g, dinv, b)


@jax.jit
def kernel(x, edge_index, W1, b1, W2, b2):
    n, _ = x.shape
    e = edge_index.shape[1]

    blk_edges = 16 * CHUNK
    per_w = ((e + NW * blk_edges - 1) // (NW * blk_edges)) * blk_edges
    e_pad = per_w * NW
    src = edge_index[0].astype(jnp.int32)
    dst = edge_index[1].astype(jnp.int32)
    pad = jnp.full((e_pad - e,), n, jnp.int32)
    nch = per_w // CHUNK
    src = jnp.concatenate([src, pad]).reshape(NW, nch, CHUNK)
    dst = jnp.concatenate([dst, pad]).reshape(NW, nch, CHUNK)
    src_e = src.reshape(NW, nch // 16, 16, CHUNK)
    dst_e = dst.reshape(NW, nch // 16, 16, CHUNK)

    x_pad = jnp.pad(x, ((0, N_PAD - n), (0, 0)))
    b1r = b1.reshape(1, D)
    b2r = b2.reshape(1, D)

    deg_parts = _make_deg_kernel(e_pad)(dst)

    edge_kernel = _make_edge_kernel(e_pad)

    g1, dinv = _mm_scale(deg_parts, x_pad, W1)
    acc1 = edge_kernel(src_e, dst_e, g1)

    g2 = _mid(acc1[0], acc1[1], g1, dinv, b1r, W2, n)
    acc2 = edge_kernel(src_e, dst_e, g2)

    out = _final(acc2[0], acc2[1], g2, dinv, b2r)
    return out[:n]


# speedup vs baseline: 1.1043x; 1.1043x over previous
import functools

import jax
import jax.numpy as jnp
from jax import lax
from jax.experimental import pallas as pl
from jax.experimental.pallas import tpu as pltpu
from jax.experimental.pallas import tpu_sc as plsc

NC, NS, L = 2, 16, 16
NW = NC * NS
D = 128
CHUNK = 128
N_PAD = 10240
BM = 512


def _make_deg_kernel(e_pad):
    per_w = e_pad // NW
    nch = per_w // CHUNK
    mesh = plsc.VectorSubcoreMesh(core_axis_name="c", subcore_axis_name="s")

    @functools.partial(
        pl.kernel,
        mesh=mesh,
        out_type=jax.ShapeDtypeStruct((NW, N_PAD), jnp.float32),
        compiler_params=pltpu.CompilerParams(needs_layout_passes=False),
        scratch_types=[
            pltpu.VMEM((nch, CHUNK), jnp.int32),
            pltpu.VMEM((N_PAD,), jnp.float32),
        ],
    )
    def deg_kernel(dst_hbm, out_hbm, idx_v, hist_v):
        c = lax.axis_index("c")
        s = lax.axis_index("s")
        wid = c * NS + s

        zeros16 = jnp.zeros((L,), jnp.float32)
        ones16 = jnp.ones((L,), jnp.float32)

        pltpu.sync_copy(dst_hbm.at[wid], idx_v)

        def zero_body(i, _):
            hist_v[pl.ds(pl.multiple_of(i * L, L), L)] = zeros16
            return 0

        lax.fori_loop(0, N_PAD // L, zero_body, 0)

        def chunk_body(j, _):
            for k in range(CHUNK // L):
                didx = idx_v[j, pl.ds(k * L, L)]
                plsc.addupdate_scatter(hist_v, [didx], ones16)
            return 0

        lax.fori_loop(0, nch, chunk_body, 0)
        pltpu.sync_copy(hist_v, out_hbm.at[wid])

    return deg_kernel


def _make_edge_kernel(e_pad):
    per_w = e_pad // NW
    nch = per_w // CHUNK
    ib = 16
    assert nch % ib == 0
    nblk = nch // ib
    zr = 16
    assert (N_PAD // NS) % zr == 0
    mesh = plsc.VectorSubcoreMesh(core_axis_name="c", subcore_axis_name="s")

    @functools.partial(
        pl.kernel,
        mesh=mesh,
        out_type=jax.ShapeDtypeStruct((NC, N_PAD, D), jnp.float32),
        compiler_params=pltpu.CompilerParams(needs_layout_passes=False),
        scratch_types=[
            pltpu.VMEM((2, ib, CHUNK), jnp.int32),
            pltpu.VMEM((2, ib, CHUNK), jnp.int32),
            pltpu.VMEM((4, CHUNK // 2, D), jnp.float32),
            pltpu.VMEM((zr, D), jnp.float32),
            pltpu.VMEM_SHARED((N_PAD, D), jnp.float32),
            pltpu.SemaphoreType.DMA,
            pltpu.SemaphoreType.DMA,
            pltpu.SemaphoreType.DMA,
            pltpu.SemaphoreType.DMA,
            pltpu.SemaphoreType.DMA,
        ],
    )
    def edge_kernel(src_hbm, dst_hbm, g_hbm, out_hbm,
                    sidx, didx, rows, zrow, acc,
                    sem0, sem1, sem2, sem3, isem):
        sems = (sem0, sem1, sem2, sem3)
        c = lax.axis_index("c")
        s = lax.axis_index("s")
        wid = c * NS + s

        pltpu.sync_copy(src_hbm.at[wid, 0], sidx.at[0])
        pltpu.sync_copy(dst_hbm.at[wid, 0], didx.at[0])

        zeros16 = jnp.zeros((L,), jnp.float32)

        def zrow_body(r, _):
            for k in range(D // L):
                zrow[r, pl.ds(k * L, L)] = zeros16
            return 0

        lax.fori_loop(0, zr, zrow_body, 0)

        rows_per_tile = N_PAD // NS

        def zcopy_body(i, _):
            off = pl.multiple_of(s * rows_per_tile + i * zr, zr)
            pltpu.sync_copy(zrow, acc.at[pl.ds(off, zr)])
            return 0

        lax.fori_loop(0, rows_per_tile // zr, zcopy_body, 0)
        plsc.subcore_barrier()

        def blk_body(n, _):
            bb = lax.rem(n, 2)

            @pl.when(n + 1 < nblk)
            def _():
                pltpu.async_copy(src_hbm.at[wid, n + 1], sidx.at[1 - bb],
                                 isem)
                pltpu.async_copy(dst_hbm.at[wid, n + 1], didx.at[1 - bb],
                                 isem)

            H = CHUNK // 2
            nsub = 2 * ib

            def gidx(bb2, j4, b):
                cj = j4 * 2 + b // 2
                off = (b % 2) * H
                return (bb2, cj, pl.ds(off, H))

            for p in range(3):
                pltpu.async_copy(g_hbm.at[sidx.at[gidx(bb, 0, p)]],
                                 rows.at[p], sems[p])

            def quad_body(j4, _):
                for b in range(4):
                    j = j4 * 4 + b
                    nb = (b + 3) % 4

                    pltpu.make_async_copy(
                        g_hbm.at[sidx.at[gidx(bb, j4, b)]],
                        rows.at[b], sems[b]).wait()

                    @pl.when(j + 3 < nsub)
                    def _():
                        pltpu.async_copy(
                            g_hbm.at[sidx.at[gidx(bb, j4, b + 3)]],
                            rows.at[nb], sems[nb])

                    pltpu.sync_copy(rows.at[b],
                                    acc.at[didx.at[gidx(bb, j4, b)]],
                                    add=True)
                return 0

            lax.fori_loop(0, nsub // 4, quad_body, 0)

            @pl.when(n + 1 < nblk)
            def _():
                pltpu.make_async_copy(src_hbm.at[wid, n + 1],
                                      sidx.at[1 - bb], isem).wait()
                pltpu.make_async_copy(dst_hbm.at[wid, n + 1],
                                      didx.at[1 - bb], isem).wait()
            return 0

        lax.fori_loop(0, nblk, blk_body, 0)
        plsc.subcore_barrier()

        out_off = pl.multiple_of(s * rows_per_tile, rows_per_tile)
        pltpu.sync_copy(acc.at[pl.ds(out_off, rows_per_tile)],
                        out_hbm.at[c, pl.ds(out_off, rows_per_tile)])

    return edge_kernel


def _mm_body(x_ref, w_ref, h_ref):
    h_ref[...] = jnp.dot(x_ref[...], w_ref[...],
                         preferred_element_type=jnp.float32)


def _mm(x, w):
    grid = (N_PAD // BM,)
    return pl.pallas_call(
        _mm_body,
        grid=grid,
        in_specs=[
            pl.BlockSpec((BM, D), lambda i: (i, 0)),
            pl.BlockSpec((D, D), lambda i: (0, 0)),
        ],
        out_specs=pl.BlockSpec((BM, D), lambda i: (i, 0)),
        out_shape=jax.ShapeDtypeStruct((N_PAD, D), jnp.float32),
    )(x, w)


def _scale_body(deg_ref, h_ref, g_ref, dinv_ref):
    deg = jnp.sum(deg_ref[...], axis=0) + 1.0
    dinv = lax.rsqrt(deg)
    g_ref[...] = h_ref[...] * dinv[:, None]
    dinv_ref[...] = dinv


def _scale(deg_parts, h):
    grid = (N_PAD // BM,)
    return pl.pallas_call(
        _scale_body,
        grid=grid,
        in_specs=[
            pl.BlockSpec((NW, BM), lambda i: (0, i)),
            pl.BlockSpec((BM, D), lambda i: (i, 0)),
        ],
        out_specs=[
            pl.BlockSpec((BM, D), lambda i: (i, 0)),
            pl.BlockSpec((BM,), lambda i: (i,)),
        ],
        out_shape=[
            jax.ShapeDtypeStruct((N_PAD, D), jnp.float32),
            jax.ShapeDtypeStruct((N_PAD,), jnp.float32),
        ],
    )(deg_parts, h)


def _mid_body(n, a0_ref, a1_ref, g_ref, dinv_ref, b_ref, w_ref, g2_ref):
    dinv = dinv_ref[...]
    acc = a0_ref[...] + a1_ref[...] + g_ref[...]
    h = jnp.maximum(acc * dinv[:, None] + b_ref[...], 0.0)
    g2 = jnp.dot(h, w_ref[...],
                 preferred_element_type=jnp.float32) * dinv[:, None]
    row = (pl.program_id(0) * BM
           + lax.broadcasted_iota(jnp.int32, (BM, D), 0))
    g2_ref[...] = jnp.where(row < n, g2, 0.0)


def _mid(a0, a1, g, dinv, b, w, n):
    grid = (N_PAD // BM,)
    return pl.pallas_call(
        functools.partial(_mid_body, n),
        grid=grid,
        in_specs=[
            pl.BlockSpec((BM, D), lambda i: (i, 0)),
            pl.BlockSpec((BM, D), lambda i: (i, 0)),
            pl.BlockSpec((BM, D), lambda i: (i, 0)),
            pl.BlockSpec((BM,), lambda i: (i,)),
            pl.BlockSpec((1, D), lambda i: (0, 0)),
            pl.BlockSpec((D, D), lambda i: (0, 0)),
        ],
        out_specs=pl.BlockSpec((BM, D), lambda i: (i, 0)),
        out_shape=jax.ShapeDtypeStruct((N_PAD, D), jnp.float32),
    )(a0, a1, g, dinv, b, w)


def _final_body(a0_ref, a1_ref, g_ref, dinv_ref, b_ref, out_ref):
    dinv = dinv_ref[...]
    acc = a0_ref[...] + a1_ref[...] + g_ref[...]
    out_ref[...] = acc * dinv[:, None] + b_ref[...]


def _final(a0, a1, g, dinv, b):
    grid = (N_PAD // BM,)
    return pl.pallas_call(
        _final_body,
        grid=grid,
        in_specs=[
            pl.BlockSpec((BM, D), lambda i: (i, 0)),
            pl.BlockSpec((BM, D), lambda i: (i, 0)),
            pl.BlockSpec((BM, D), lambda i: (i, 0)),
            pl.BlockSpec((BM,), lambda i: (i,)),
            pl.BlockSpec((1, D), lambda i: (0, 0)),
        ],
        out_specs=pl.BlockSpec((BM, D), lambda i: (i, 0)),
        out_shape=jax.ShapeDtypeStruct((N_PAD, D), jnp.float32),
    )(a0, a1, g, dinv, b)


@jax.jit
def kernel(x, edge_index, W1, b1, W2, b2):
    n, _ = x.shape
    e = edge_index.shape[1]

    blk_edges = 16 * CHUNK
    per_w = ((e + NW * blk_edges - 1) // (NW * blk_edges)) * blk_edges
    e_pad = per_w * NW
    src = edge_index[0].astype(jnp.int32)
    dst = edge_index[1].astype(jnp.int32)
    pad = jnp.full((e_pad - e,), n, jnp.int32)
    nch = per_w // CHUNK
    src = jnp.concatenate([src, pad]).reshape(NW, nch, CHUNK)
    dst = jnp.concatenate([dst, pad]).reshape(NW, nch, CHUNK)
    src_e = src.reshape(NW, nch // 16, 16, CHUNK)
    dst_e = dst.reshape(NW, nch // 16, 16, CHUNK)

    x_pad = jnp.pad(x, ((0, N_PAD - n), (0, 0)))
    b1r = b1.reshape(1, D)
    b2r = b2.reshape(1, D)

    deg_parts = _make_deg_kernel(e_pad)(dst)

    edge_kernel = _make_edge_kernel(e_pad)

    h1 = _mm(x_pad, W1)
    g1, dinv = _scale(deg_parts, h1)
    acc1 = edge_kernel(src_e, dst_e, g1)

    g2 = _mid(acc1[0], acc1[1], g1, dinv, b1r, W2, n)
    acc2 = edge_kernel(src_e, dst_e, g2)

    out = _final(acc2[0], acc2[1], g2, dinv, b2r)
    return out[:n]
